# Initial kernel scaffold; baseline (speedup 1.0000x reference)
import jax
import jax.numpy as jnp
from jax.experimental import pallas as pl


def kernel(x, w_gate, W1, b1, W2, b2):
    raise NotImplementedError("write your pallas kernel here")


# speedup vs baseline: 4.1657x; 4.1657x over previous
import functools

import jax
import jax.numpy as jnp
from jax.experimental import pallas as pl
from jax.experimental.pallas import tpu as pltpu

_NUM_EXPERTS = 8
_TOP_K = 2
_D = 768
_N = 8192
_CAP = 4096
_LOSS_COEF = 0.01
_TB = 512
_FB = 512
_LANES = 128


def _router_kernel(x_ref, wg_ref, w_out_ref, loss_ref,
                   cnt_ref, imp_ref, load_ref):
    i = pl.program_id(0)
    nblocks = pl.num_programs(0)

    @pl.when(i == 0)
    def _init():
        cnt_ref[...] = jnp.zeros_like(cnt_ref)
        imp_ref[...] = jnp.zeros_like(imp_ref)
        load_ref[...] = jnp.zeros_like(load_ref)

    xb = x_ref[...]
    wg = wg_ref[...]
    logits = jax.lax.dot_general(
        xb, wg, (((1,), (0,)), ((), ())),
        preferred_element_type=jnp.float32)
    lane = jax.lax.broadcasted_iota(jnp.int32, logits.shape, 1)
    valid = lane < _NUM_EXPERTS
    neg = jnp.float32(-1e30)
    logits = jnp.where(valid, logits, neg)

    m0 = jnp.max(logits, axis=1, keepdims=True)
    is0 = logits == m0
    idx0 = jnp.min(jnp.where(is0, lane, _LANES), axis=1, keepdims=True)
    oh0 = lane == idx0
    logits1 = jnp.where(oh0, neg, logits)
    m1 = jnp.max(logits1, axis=1, keepdims=True)
    is1 = logits1 == m1
    idx1 = jnp.min(jnp.where(is1, lane, _LANES), axis=1, keepdims=True)
    oh1 = lane == idx1

    e1 = jnp.exp(m1 - m0)
    denom = 1.0 + e1
    g0 = 1.0 / denom
    g1 = e1 / denom

    oh0f = oh0.astype(jnp.float32)
    oh1f = oh1.astype(jnp.float32)
    gates = g0 * oh0f + g1 * oh1f
    imp_ref[...] += jnp.sum(gates, axis=0, keepdims=True)
    load_ref[...] += jnp.sum((gates > 0).astype(jnp.float32), axis=0,
                             keepdims=True)

    onehot2 = oh0f + oh1f
    row = jax.lax.broadcasted_iota(jnp.int32, (_TB, _TB), 0)
    col = jax.lax.broadcasted_iota(jnp.int32, (_TB, _TB), 1)
    tri = (col < row).astype(jnp.bfloat16)
    csum = jax.lax.dot_general(
        tri, onehot2.astype(jnp.bfloat16), (((1,), (0,)), ((), ())),
        preferred_element_type=jnp.float32)
    pos_before = cnt_ref[...] + csum
    pos0 = jnp.sum(pos_before * oh0f, axis=1, keepdims=True)
    pos1 = jnp.sum(pos_before * oh1f, axis=1, keepdims=True)
    keep0 = (pos0 < _CAP).astype(jnp.float32)
    keep1 = (pos1 < _CAP).astype(jnp.float32)
    w_out_ref[...] = (g0 * keep0) * oh0f + (g1 * keep1) * oh1f
    cnt_ref[...] += jnp.sum(onehot2, axis=0, keepdims=True)

    @pl.when(i == nblocks - 1)
    def _loss():
        inv_e = 1.0 / _NUM_EXPERTS
        vmask = (jax.lax.broadcasted_iota(jnp.int32, (1, _LANES), 1)
                 < _NUM_EXPERTS).astype(jnp.float32)

        def cv2(v):
            mean = jnp.sum(v * vmask) * inv_e
            var = jnp.sum((v - mean) ** 2 * vmask) * inv_e
            return var / (mean * mean + 1e-10)

        lv = (cv2(imp_ref[...]) + cv2(load_ref[...])) * _LOSS_COEF
        loss_ref[...] = jnp.full((1, 1), lv, jnp.float32)


def _expert_kernel(x_ref, w_ref, w1_ref, b1_ref, w2_ref, b2_ref, out_ref):
    xb = x_ref[...]
    wb = w_ref[...]
    acc = xb
    xbh = xb.astype(jnp.bfloat16)
    for e in range(_NUM_EXPERTS):
        h = jax.lax.dot_general(
            xbh, w1_ref[e].astype(jnp.bfloat16), (((1,), (0,)), ((), ())),
            preferred_element_type=jnp.float32) + b1_ref[e][None, :]
        h = jax.nn.gelu(h)
        y = jax.lax.dot_general(
            h.astype(jnp.bfloat16), w2_ref[e].astype(jnp.bfloat16),
            (((1,), (0,)), ((), ())),
            preferred_element_type=jnp.float32) + b2_ref[e][None, :]
        acc = acc + wb[:, e:e + 1] * y
    out_ref[...] = acc


@jax.jit
def kernel(x, w_gate, W1, b1, W2, b2):
    wg_pad = jnp.zeros((_D, _LANES), jnp.float32).at[:, :_NUM_EXPERTS].set(
        w_gate)

    w_mask, loss = pl.pallas_call(
        _router_kernel,
        grid=(_N // _TB,),
        in_specs=[
            pl.BlockSpec((_TB, _D), lambda i: (i, 0)),
            pl.BlockSpec((_D, _LANES), lambda i: (0, 0)),
        ],
        out_specs=[
            pl.BlockSpec((_TB, _LANES), lambda i: (i, 0)),
            pl.BlockSpec((1, 1), lambda i: (0, 0)),
        ],
        out_shape=[
            jax.ShapeDtypeStruct((_N, _LANES), jnp.float32),
            jax.ShapeDtypeStruct((1, 1), jnp.float32),
        ],
        scratch_shapes=[
            pltpu.VMEM((1, _LANES), jnp.float32),
            pltpu.VMEM((1, _LANES), jnp.float32),
            pltpu.VMEM((1, _LANES), jnp.float32),
        ],
    )(x, wg_pad)

    out = pl.pallas_call(
        _expert_kernel,
        grid=(_N // _FB,),
        in_specs=[
            pl.BlockSpec((_FB, _D), lambda i: (i, 0)),
            pl.BlockSpec((_FB, _LANES), lambda i: (i, 0)),
            pl.BlockSpec((_NUM_EXPERTS, _D, _D), lambda i: (0, 0, 0)),
            pl.BlockSpec((_NUM_EXPERTS, _D), lambda i: (0, 0)),
            pl.BlockSpec((_NUM_EXPERTS, _D, _D), lambda i: (0, 0, 0)),
            pl.BlockSpec((_NUM_EXPERTS, _D), lambda i: (0, 0)),
        ],
        out_specs=pl.BlockSpec((_FB, _D), lambda i: (i, 0)),
        out_shape=jax.ShapeDtypeStruct((_N, _D), jnp.float32),
    )(x, w_mask, W1, b1, W2, b2)

    return (out, jnp.reshape(loss, ()))
